# baseline (speedup 1.0000x reference)
import functools

import jax
import jax.numpy as jnp
from jax import lax
from jax.experimental import pallas as pl
from jax.experimental.pallas import tpu as pltpu
from jax.experimental.pallas import tpu_sc as plsc

N = 10000
E = 320000
D = 128
H_ACTOR = 256
G = 64

NC = 2
NS = 16
CHUNK = 64
E_PAD = 327680
EPT = E_PAD // (NS * NC)
TPT = EPT // CHUNK
QCH = TPT // 4

N_ACC = 10112
N_HIST = 10240
R = 1000
NB = N // R

_SC_MESH = plsc.VectorSubcoreMesh(core_axis_name="c", subcore_axis_name="s")


@functools.partial(
    pl.kernel,
    out_type=jax.ShapeDtypeStruct((NC, N_HIST), jnp.float32),
    scratch_types=[
        pltpu.VMEM((TPT, CHUNK), jnp.int32),
        pltpu.VMEM((CHUNK,), jnp.float32),
        pltpu.VMEM_SHARED((N_HIST,), jnp.float32),
        pltpu.SemaphoreType.DMA,
        pltpu.SemaphoreType.DMA,
    ],
    mesh=_SC_MESH,
)
def _deg_kernel(dst_hbm, zeros_hbm, out_hbm, dst_t, ones_v, hist_sh,
                sa0, sa1):
    c = lax.axis_index("c")
    s = lax.axis_index("s")
    for j in range(CHUNK // 16):
        ones_v[pl.ds(j * 16, 16)] = jnp.full((16,), 1.0, dtype=jnp.float32)
    tph = N_HIST // NS
    pltpu.sync_copy(zeros_hbm.at[pl.ds(s * tph, tph)],
                    hist_sh.at[pl.ds(s * tph, tph)])
    cb = (c * NS + s) * TPT
    pltpu.sync_copy(dst_hbm.at[pl.ds(cb, TPT)], dst_t)
    plsc.subcore_barrier()

    pltpu.async_copy(ones_v, hist_sh.at[dst_t.at[0]], sa0, add=True)
    pltpu.async_copy(ones_v, hist_sh.at[dst_t.at[1]], sa1, add=True)

    def body(i, carry):
        j = 2 * i + 2
        pltpu.make_async_copy(ones_v, hist_sh.at[dst_t.at[0]], sa0).wait()
        pltpu.async_copy(ones_v, hist_sh.at[dst_t.at[j]], sa0, add=True)
        pltpu.make_async_copy(ones_v, hist_sh.at[dst_t.at[0]], sa1).wait()
        pltpu.async_copy(ones_v, hist_sh.at[dst_t.at[j + 1]], sa1, add=True)
        return carry

    lax.fori_loop(0, (TPT - 2) // 2, body, 0)
    pltpu.make_async_copy(ones_v, hist_sh.at[dst_t.at[0]], sa0).wait()
    pltpu.make_async_copy(ones_v, hist_sh.at[dst_t.at[0]], sa1).wait()
    plsc.subcore_barrier()
    pltpu.sync_copy(hist_sh.at[pl.ds(s * tph, tph)],
                    out_hbm.at[c, pl.ds(s * tph, tph)])


@functools.partial(
    pl.kernel,
    out_type=jax.ShapeDtypeStruct((NC, N_ACC, D), jnp.float32),
    scratch_types=[
        pltpu.VMEM((QCH, CHUNK), jnp.int32),
        pltpu.VMEM((QCH, CHUNK), jnp.int32),
        pltpu.VMEM((CHUNK, D), jnp.float32),
        pltpu.VMEM((CHUNK, D), jnp.float32),
        pltpu.VMEM((CHUNK, D), jnp.float32),
        pltpu.VMEM((CHUNK, D), jnp.float32),
        pltpu.VMEM_SHARED((N_ACC, D), jnp.float32),
        pltpu.SemaphoreType.DMA,
        pltpu.SemaphoreType.DMA,
        pltpu.SemaphoreType.DMA,
        pltpu.SemaphoreType.DMA,
        pltpu.SemaphoreType.DMA,
        pltpu.SemaphoreType.DMA,
        pltpu.SemaphoreType.DMA,
        pltpu.SemaphoreType.DMA,
    ],
    mesh=_SC_MESH,
)
def _edge_kernel(y_hbm, src_hbm, dst_hbm, zeros_hbm, out_hbm,
                 src_t, dst_t, r0, r1, r2, r3, acc_sh,
                 g0, g1, g2, g3, s0, s1, s2, s3):
    c = lax.axis_index("c")
    s = lax.axis_index("s")
    rpt = N_ACC // NS

    @pl.when(c == 0)
    def _():
        pltpu.sync_copy(y_hbm.at[pl.ds(s * rpt, rpt)],
                        acc_sh.at[pl.ds(s * rpt, rpt)])

    @pl.when(c == 1)
    def _():
        pltpu.sync_copy(zeros_hbm.at[pl.ds(s * rpt, rpt)],
                        acc_sh.at[pl.ds(s * rpt, rpt)])

    plsc.subcore_barrier()

    rows = [r0, r1, r2, r3]
    gs = [g0, g1, g2, g3]
    ss = [s0, s1, s2, s3]

    def gather(j, b):
        pltpu.async_copy(y_hbm.at[src_t.at[j]], rows[b], gs[b])

    def wait_gather(b):
        pltpu.make_async_copy(y_hbm.at[src_t.at[0]], rows[b], gs[b]).wait()

    def scatter(j, b):
        pltpu.async_copy(rows[b], acc_sh.at[dst_t.at[j]], ss[b], add=True)

    def wait_scatter(b):
        pltpu.make_async_copy(rows[b], acc_sh.at[dst_t.at[0]], ss[b]).wait()

    for q in range(4):
        cb = (c * NS + s) * TPT + q * QCH
        pltpu.sync_copy(src_hbm.at[pl.ds(cb, QCH)], src_t)
        pltpu.sync_copy(dst_hbm.at[pl.ds(cb, QCH)], dst_t)

        gather(0, 0)
        gather(1, 1)
        wait_gather(0)
        scatter(0, 0)
        gather(2, 2)
        wait_gather(1)
        scatter(1, 1)
        gather(3, 3)

        def body(i, carry):
            for u in range(4):
                j = 4 * i + 2 + u
                b = (2 + u) % 4
                wait_gather(b)
                scatter(j, b)
                wait_scatter((b + 2) % 4)
                gather(j + 2, (b + 2) % 4)
            return carry

        lax.fori_loop(0, (QCH - 4) // 4, body, 0)
        wait_gather(2)
        scatter(QCH - 2, 2)
        wait_gather(3)
        scatter(QCH - 1, 3)
        wait_scatter(0)
        wait_scatter(1)
        wait_scatter(2)
        wait_scatter(3)

    plsc.subcore_barrier()
    pltpu.sync_copy(acc_sh.at[pl.ds(s * rpt, rpt)],
                    out_hbm.at[c, pl.ds(s * rpt, rpt)])


def _tc0_body(histT_ref, x_ref, w_ref, y_ref, dinv_ref):
    deg = histT_ref[:, 0:1] + histT_ref[:, 1:2] + 1.0
    dinv = lax.rsqrt(deg)
    dinv_ref[...] = dinv
    y_ref[...] = dinv * jnp.dot(x_ref[...], w_ref[...],
                                preferred_element_type=jnp.float32)


_tc0_call = pl.pallas_call(
    _tc0_body,
    grid=(NB,),
    in_specs=[
        pl.BlockSpec((R, 2), lambda i: (i, 0)),
        pl.BlockSpec((R, D), lambda i: (i, 0)),
        pl.BlockSpec((D, D), lambda i: (0, 0)),
    ],
    out_specs=[
        pl.BlockSpec((R, D), lambda i: (i, 0)),
        pl.BlockSpec((R, 1), lambda i: (i, 0)),
    ],
    out_shape=[
        jax.ShapeDtypeStruct((N_ACC, D), jnp.float32),
        jax.ShapeDtypeStruct((N, 1), jnp.float32),
    ],
)


def _tcl_body(acc_ref, dinv_ref, b_ref, w_ref, out_ref):
    d = dinv_ref[...]
    h = jnp.maximum(d * (acc_ref[0] + acc_ref[1]) + b_ref[...], 0.0)
    out_ref[...] = d * jnp.dot(h, w_ref[...],
                               preferred_element_type=jnp.float32)


_tcl_call = pl.pallas_call(
    _tcl_body,
    grid=(NB,),
    in_specs=[
        pl.BlockSpec((NC, R, D), lambda i: (0, i, 0)),
        pl.BlockSpec((R, 1), lambda i: (i, 0)),
        pl.BlockSpec((1, D), lambda i: (0, 0)),
        pl.BlockSpec((D, D), lambda i: (0, 0)),
    ],
    out_specs=pl.BlockSpec((R, D), lambda i: (i, 0)),
    out_shape=jax.ShapeDtypeStruct((N_ACC, D), jnp.float32),
)


def _tcf_body(acc_ref, dinv_ref, b_ref, nw_ref, nb_ref, batch_ref,
              w1_ref, b1_ref, w2_ref, b2_ref,
              np_ref, pooled_ref, cnt_ref, fea_ref):
    i = pl.program_id(0)
    d = dinv_ref[...]
    h = jnp.maximum(d * (acc_ref[0] + acc_ref[1]) + b_ref[...], 0.0)
    np_ref[...] = jnp.dot(h, nw_ref[...],
                          preferred_element_type=jnp.float32) + nb_ref[0, 0]
    gid = lax.broadcasted_iota(jnp.int32, (R, G), 1)
    m = (batch_ref[...] == gid).astype(jnp.float32)
    dn = (((0,), (0,)), ((), ()))
    part = lax.dot_general(m, h, dn, preferred_element_type=jnp.float32)
    cpart = lax.dot_general(m, jnp.ones((R, 1), jnp.float32), dn,
                            preferred_element_type=jnp.float32)

    @pl.when(i == 0)
    def _():
        pooled_ref[...] = part
        cnt_ref[...] = cpart

    @pl.when(i > 0)
    def _():
        pooled_ref[...] = pooled_ref[...] + part
        cnt_ref[...] = cnt_ref[...] + cpart

    @pl.when(i == NB - 1)
    def _():
        pooled = pooled_ref[...] / jnp.maximum(cnt_ref[...], 1.0)
        f = jnp.maximum(
            jnp.dot(pooled, w1_ref[...],
                    preferred_element_type=jnp.float32) + b1_ref[...], 0.0)
        fea_ref[...] = jnp.dot(f, w2_ref[...],
                               preferred_element_type=jnp.float32) + b2_ref[...]


_tcf_call = pl.pallas_call(
    _tcf_body,
    grid=(NB,),
    in_specs=[
        pl.BlockSpec((NC, R, D), lambda i: (0, i, 0)),
        pl.BlockSpec((R, 1), lambda i: (i, 0)),
        pl.BlockSpec((1, D), lambda i: (0, 0)),
        pl.BlockSpec((D, 1), lambda i: (0, 0)),
        pl.BlockSpec((1, 1), lambda i: (0, 0)),
        pl.BlockSpec((R, 1), lambda i: (i, 0)),
        pl.BlockSpec((D, H_ACTOR), lambda i: (0, 0)),
        pl.BlockSpec((1, H_ACTOR), lambda i: (0, 0)),
        pl.BlockSpec((H_ACTOR, D), lambda i: (0, 0)),
        pl.BlockSpec((1, D), lambda i: (0, 0)),
    ],
    out_specs=[
        pl.BlockSpec((R, 1), lambda i: (i, 0)),
        pl.BlockSpec((G, D), lambda i: (0, 0)),
        pl.BlockSpec((G, 1), lambda i: (0, 0)),
        pl.BlockSpec((G, D), lambda i: (0, 0)),
    ],
    out_shape=[
        jax.ShapeDtypeStruct((N, 1), jnp.float32),
        jax.ShapeDtypeStruct((G, D), jnp.float32),
        jax.ShapeDtypeStruct((G, 1), jnp.float32),
        jax.ShapeDtypeStruct((G, D), jnp.float32),
    ],
)


def kernel(x, edge_index, batch, conv_W0, conv_b0, conv_W1, conv_b1,
           conv_W2, conv_b2, node_W, node_b, fea1_W, fea1_b, fea2_W, fea2_b):
    pad = E_PAD - E
    pr = jnp.arange(pad, dtype=jnp.int32)
    srcp = jnp.concatenate(
        [edge_index[0], pr % N]).reshape(-1, CHUNK)
    dstp = jnp.concatenate(
        [edge_index[1], N + pr % (N_ACC - N)]).reshape(-1, CHUNK)
    zeros_hist = jnp.zeros((N_HIST,), jnp.float32)
    zeros_acc = jnp.zeros((N_ACC, D), jnp.float32)

    hist = _deg_kernel(dstp, zeros_hist)
    histT = hist.T

    y0, dinv = _tc0_call(histT, x, conv_W0)
    acc = _edge_kernel(y0, srcp, dstp, zeros_acc)
    y1 = _tcl_call(acc, dinv, conv_b0.reshape(1, D), conv_W1)
    acc = _edge_kernel(y1, srcp, dstp, zeros_acc)
    y2 = _tcl_call(acc, dinv, conv_b1.reshape(1, D), conv_W2)
    acc = _edge_kernel(y2, srcp, dstp, zeros_acc)

    node_col, _, _, fea_prob = _tcf_call(
        acc, dinv, conv_b2.reshape(1, D), node_W,
        node_b.reshape(1, 1), batch.reshape(N, 1),
        fea1_W, fea1_b.reshape(1, H_ACTOR), fea2_W, fea2_b.reshape(1, D))
    return (node_col.reshape(N), fea_prob)


# speedup vs baseline: 1.1279x; 1.1279x over previous
import functools

import jax
import jax.numpy as jnp
from jax import lax
from jax.experimental import pallas as pl
from jax.experimental.pallas import tpu as pltpu
from jax.experimental.pallas import tpu_sc as plsc

N = 10000
E = 320000
D = 128
H_ACTOR = 256
G = 64

NC = 2
NS = 16
CHUNK = 128
NCHUNK = 80
E_PAD = NCHUNK * CHUNK * NS * NC
EPT = NCHUNK * CHUNK
EPC = EPT * NS
NCH0 = 80
NCH1 = 2 * NCHUNK - NCH0
_MAXCH = max(NCH0, NCH1)

N_ACC = 10112
N_HIST = 10240
R = 1000
NB = N // R

_SC_MESH = plsc.VectorSubcoreMesh(core_axis_name="c", subcore_axis_name="s")


@functools.partial(
    pl.kernel,
    out_type=jax.ShapeDtypeStruct((NC, N_HIST), jnp.float32),
    scratch_types=[
        pltpu.VMEM((NCHUNK, CHUNK), jnp.int32),
        pltpu.VMEM((CHUNK,), jnp.float32),
        pltpu.VMEM_SHARED((N_HIST,), jnp.float32),
        pltpu.SemaphoreType.DMA,
        pltpu.SemaphoreType.DMA,
    ],
    mesh=_SC_MESH,
)
def _deg_kernel(dst_hbm, zeros_hbm, out_hbm, dst_t, ones_v, hist_sh,
                sa0, sa1):
    c = lax.axis_index("c")
    s = lax.axis_index("s")
    for j in range(CHUNK // 16):
        ones_v[pl.ds(j * 16, 16)] = jnp.full((16,), 1.0, dtype=jnp.float32)
    tph = N_HIST // NS
    pltpu.sync_copy(zeros_hbm.at[pl.ds(s * tph, tph)],
                    hist_sh.at[pl.ds(s * tph, tph)])
    cb = (c * NS + s) * NCHUNK
    pltpu.sync_copy(dst_hbm.at[pl.ds(cb, NCHUNK)], dst_t)
    plsc.subcore_barrier()

    pltpu.async_copy(ones_v, hist_sh.at[dst_t.at[0]], sa0, add=True)
    pltpu.async_copy(ones_v, hist_sh.at[dst_t.at[1]], sa1, add=True)

    def body(i, carry):
        j = 2 * i + 2
        pltpu.make_async_copy(ones_v, hist_sh.at[dst_t.at[0]], sa0).wait()
        pltpu.async_copy(ones_v, hist_sh.at[dst_t.at[j]], sa0, add=True)
        pltpu.make_async_copy(ones_v, hist_sh.at[dst_t.at[0]], sa1).wait()
        pltpu.async_copy(ones_v, hist_sh.at[dst_t.at[j + 1]], sa1, add=True)
        return carry

    lax.fori_loop(0, (NCHUNK - 2) // 2, body, 0)
    pltpu.make_async_copy(ones_v, hist_sh.at[dst_t.at[0]], sa0).wait()
    pltpu.make_async_copy(ones_v, hist_sh.at[dst_t.at[0]], sa1).wait()
    plsc.subcore_barrier()
    pltpu.sync_copy(hist_sh.at[pl.ds(s * tph, tph)],
                    out_hbm.at[c, pl.ds(s * tph, tph)])


@functools.partial(
    pl.kernel,
    out_type=jax.ShapeDtypeStruct((NC, N_ACC, D), jnp.float32),
    scratch_types=[
        pltpu.VMEM((_MAXCH // 2, CHUNK), jnp.int32),
        pltpu.VMEM((_MAXCH // 2, CHUNK), jnp.int32),
        pltpu.VMEM((CHUNK, D), jnp.float32),
        pltpu.VMEM((CHUNK, D), jnp.float32),
        pltpu.VMEM_SHARED((N_ACC, D), jnp.float32),
        pltpu.SemaphoreType.DMA,
        pltpu.SemaphoreType.DMA,
        pltpu.SemaphoreType.DMA,
        pltpu.SemaphoreType.DMA,
    ],
    mesh=_SC_MESH,
)
def _edge_kernel(y_hbm, src_hbm, dst_hbm, zeros_hbm, out_hbm,
                 src_t, dst_t, rows0, rows1, acc_sh, gs0, gs1, ss0, ss1):
    c = lax.axis_index("c")
    s = lax.axis_index("s")
    rpt = N_ACC // NS

    @pl.when(c == 0)
    def _():
        pltpu.sync_copy(y_hbm.at[pl.ds(s * rpt, rpt)],
                        acc_sh.at[pl.ds(s * rpt, rpt)])

    @pl.when(c == 1)
    def _():
        pltpu.sync_copy(zeros_hbm.at[pl.ds(s * rpt, rpt)],
                        acc_sh.at[pl.ds(s * rpt, rpt)])

    plsc.subcore_barrier()

    def pipe(cbase, nch):
        half = nch // 2
        for h in range(2):
            cb = cbase + h * half
            pltpu.sync_copy(src_hbm.at[pl.ds(cb, half)],
                            src_t.at[pl.ds(0, half)])
            pltpu.sync_copy(dst_hbm.at[pl.ds(cb, half)],
                            dst_t.at[pl.ds(0, half)])

            pltpu.async_copy(y_hbm.at[src_t.at[0]], rows0, gs0)
            pltpu.async_copy(y_hbm.at[src_t.at[1]], rows1, gs1)
            pltpu.make_async_copy(y_hbm.at[src_t.at[0]], rows0, gs0).wait()
            pltpu.async_copy(rows0, acc_sh.at[dst_t.at[0]], ss0, add=True)

            def body(i, carry):
                j = 2 * i + 1
                pltpu.make_async_copy(rows0, acc_sh.at[dst_t.at[0]],
                                      ss0).wait()
                pltpu.async_copy(y_hbm.at[src_t.at[j + 1]], rows0, gs0)
                pltpu.make_async_copy(y_hbm.at[src_t.at[0]], rows1,
                                      gs1).wait()
                pltpu.async_copy(rows1, acc_sh.at[dst_t.at[j]], ss1,
                                 add=True)

                pltpu.make_async_copy(rows1, acc_sh.at[dst_t.at[0]],
                                      ss1).wait()
                pltpu.async_copy(y_hbm.at[src_t.at[j + 2]], rows1, gs1)
                pltpu.make_async_copy(y_hbm.at[src_t.at[0]], rows0,
                                      gs0).wait()
                pltpu.async_copy(rows0, acc_sh.at[dst_t.at[j + 1]], ss0,
                                 add=True)
                return carry

            lax.fori_loop(0, (half - 2) // 2, body, 0)
            pltpu.make_async_copy(rows0, acc_sh.at[dst_t.at[0]], ss0).wait()
            pltpu.make_async_copy(y_hbm.at[src_t.at[0]], rows1, gs1).wait()
            pltpu.async_copy(rows1, acc_sh.at[dst_t.at[half - 1]], ss1,
                             add=True).wait()

    @pl.when(c == 0)
    def _():
        pipe(s * NCH0, NCH0)

    @pl.when(c == 1)
    def _():
        pipe(NS * NCH0 + s * NCH1, NCH1)

    plsc.subcore_barrier()
    pltpu.sync_copy(acc_sh.at[pl.ds(s * rpt, rpt)],
                    out_hbm.at[c, pl.ds(s * rpt, rpt)])


def _tc0_body(histT_ref, x_ref, w_ref, y_ref, dinv_ref):
    deg = histT_ref[:, 0:1] + histT_ref[:, 1:2] + 1.0
    dinv = lax.rsqrt(deg)
    dinv_ref[...] = dinv
    y_ref[...] = dinv * jnp.dot(x_ref[...], w_ref[...],
                                preferred_element_type=jnp.float32)


_tc0_call = pl.pallas_call(
    _tc0_body,
    grid=(NB,),
    in_specs=[
        pl.BlockSpec((R, 2), lambda i: (i, 0)),
        pl.BlockSpec((R, D), lambda i: (i, 0)),
        pl.BlockSpec((D, D), lambda i: (0, 0)),
    ],
    out_specs=[
        pl.BlockSpec((R, D), lambda i: (i, 0)),
        pl.BlockSpec((R, 1), lambda i: (i, 0)),
    ],
    out_shape=[
        jax.ShapeDtypeStruct((N_ACC, D), jnp.float32),
        jax.ShapeDtypeStruct((N, 1), jnp.float32),
    ],
)


def _tcl_body(acc_ref, dinv_ref, b_ref, w_ref, out_ref):
    d = dinv_ref[...]
    h = jnp.maximum(d * (acc_ref[0] + acc_ref[1]) + b_ref[...], 0.0)
    out_ref[...] = d * jnp.dot(h, w_ref[...],
                               preferred_element_type=jnp.float32)


_tcl_call = pl.pallas_call(
    _tcl_body,
    grid=(NB,),
    in_specs=[
        pl.BlockSpec((NC, R, D), lambda i: (0, i, 0)),
        pl.BlockSpec((R, 1), lambda i: (i, 0)),
        pl.BlockSpec((1, D), lambda i: (0, 0)),
        pl.BlockSpec((D, D), lambda i: (0, 0)),
    ],
    out_specs=pl.BlockSpec((R, D), lambda i: (i, 0)),
    out_shape=jax.ShapeDtypeStruct((N_ACC, D), jnp.float32),
)


def _tcf_body(acc_ref, dinv_ref, b_ref, nw_ref, nb_ref, batch_ref,
              w1_ref, b1_ref, w2_ref, b2_ref,
              np_ref, pooled_ref, cnt_ref, fea_ref):
    i = pl.program_id(0)
    d = dinv_ref[...]
    h = jnp.maximum(d * (acc_ref[0] + acc_ref[1]) + b_ref[...], 0.0)
    np_ref[...] = jnp.dot(h, nw_ref[...],
                          preferred_element_type=jnp.float32) + nb_ref[0, 0]
    gid = lax.broadcasted_iota(jnp.int32, (R, G), 1)
    m = (batch_ref[...] == gid).astype(jnp.float32)
    dn = (((0,), (0,)), ((), ()))
    part = lax.dot_general(m, h, dn, preferred_element_type=jnp.float32)
    cpart = lax.dot_general(m, jnp.ones((R, 1), jnp.float32), dn,
                            preferred_element_type=jnp.float32)

    @pl.when(i == 0)
    def _():
        pooled_ref[...] = part
        cnt_ref[...] = cpart

    @pl.when(i > 0)
    def _():
        pooled_ref[...] = pooled_ref[...] + part
        cnt_ref[...] = cnt_ref[...] + cpart

    @pl.when(i == NB - 1)
    def _():
        pooled = pooled_ref[...] / jnp.maximum(cnt_ref[...], 1.0)
        f = jnp.maximum(
            jnp.dot(pooled, w1_ref[...],
                    preferred_element_type=jnp.float32) + b1_ref[...], 0.0)
        fea_ref[...] = jnp.dot(f, w2_ref[...],
                               preferred_element_type=jnp.float32) + b2_ref[...]


_tcf_call = pl.pallas_call(
    _tcf_body,
    grid=(NB,),
    in_specs=[
        pl.BlockSpec((NC, R, D), lambda i: (0, i, 0)),
        pl.BlockSpec((R, 1), lambda i: (i, 0)),
        pl.BlockSpec((1, D), lambda i: (0, 0)),
        pl.BlockSpec((D, 1), lambda i: (0, 0)),
        pl.BlockSpec((1, 1), lambda i: (0, 0)),
        pl.BlockSpec((R, 1), lambda i: (i, 0)),
        pl.BlockSpec((D, H_ACTOR), lambda i: (0, 0)),
        pl.BlockSpec((1, H_ACTOR), lambda i: (0, 0)),
        pl.BlockSpec((H_ACTOR, D), lambda i: (0, 0)),
        pl.BlockSpec((1, D), lambda i: (0, 0)),
    ],
    out_specs=[
        pl.BlockSpec((R, 1), lambda i: (i, 0)),
        pl.BlockSpec((G, D), lambda i: (0, 0)),
        pl.BlockSpec((G, 1), lambda i: (0, 0)),
        pl.BlockSpec((G, D), lambda i: (0, 0)),
    ],
    out_shape=[
        jax.ShapeDtypeStruct((N, 1), jnp.float32),
        jax.ShapeDtypeStruct((G, D), jnp.float32),
        jax.ShapeDtypeStruct((G, 1), jnp.float32),
        jax.ShapeDtypeStruct((G, D), jnp.float32),
    ],
)


def kernel(x, edge_index, batch, conv_W0, conv_b0, conv_W1, conv_b1,
           conv_W2, conv_b2, node_W, node_b, fea1_W, fea1_b, fea2_W, fea2_b):
    pad = E_PAD - E
    pr = jnp.arange(pad, dtype=jnp.int32)
    srcp = jnp.concatenate(
        [edge_index[0], pr % N]).reshape(-1, CHUNK)
    dstp = jnp.concatenate(
        [edge_index[1], N + pr % (N_ACC - N)]).reshape(-1, CHUNK)
    zeros_hist = jnp.zeros((N_HIST,), jnp.float32)
    zeros_acc = jnp.zeros((N_ACC, D), jnp.float32)

    hist = _deg_kernel(dstp, zeros_hist)
    histT = hist.T

    y0, dinv = _tc0_call(histT, x, conv_W0)
    acc = _edge_kernel(y0, srcp, dstp, zeros_acc)
    y1 = _tcl_call(acc, dinv, conv_b0.reshape(1, D), conv_W1)
    acc = _edge_kernel(y1, srcp, dstp, zeros_acc)
    y2 = _tcl_call(acc, dinv, conv_b1.reshape(1, D), conv_W2)
    acc = _edge_kernel(y2, srcp, dstp, zeros_acc)

    node_col, _, _, fea_prob = _tcf_call(
        acc, dinv, conv_b2.reshape(1, D), node_W,
        node_b.reshape(1, 1), batch.reshape(N, 1),
        fea1_W, fea1_b.reshape(1, H_ACTOR), fea2_W, fea2_b.reshape(1, D))
    return (node_col.reshape(N), fea_prob)
